# baseline (speedup 1.0000x reference)
import functools

import jax
import jax.numpy as jnp
from jax import lax
from jax.experimental import pallas as pl
from jax.experimental.pallas import tpu as pltpu
from jax.experimental.pallas import tpu_sc as plsc

N_ROWS = 16384
N_IN = 1278
N_OUT = 2048
L = 16

NC = 2
NS = 16
NW = NC * NS
ROWS_PER_W = N_ROWS // NW
R = 16
CHUNKS = ROWS_PER_W // R
G = N_OUT // L
NBUF = 2


def kernel(x, routing_indices, valid_mask):
    mesh = plsc.VectorSubcoreMesh(core_axis_name="c", subcore_axis_name="s")

    @functools.partial(
        pl.kernel,
        mesh=mesh,
        out_type=jax.ShapeDtypeStruct((N_ROWS, N_OUT), jnp.float32),
        compiler_params=pltpu.CompilerParams(needs_layout_passes=False),
        scratch_types=[
            pltpu.VMEM((N_OUT,), jnp.int32),
            pltpu.VMEM((N_OUT,), jnp.float32),
            pltpu.VMEM((R, N_IN), jnp.float32),
            pltpu.VMEM((R, N_IN), jnp.float32),
            pltpu.VMEM((R, N_OUT), jnp.float32),
            pltpu.VMEM((R, N_OUT), jnp.float32),
            pltpu.SemaphoreType.DMA((NBUF,)),
            pltpu.SemaphoreType.DMA((NBUF,)),
        ],
    )
    def k(x_hbm, ri_hbm, vm_hbm, out_hbm, idx_v, vm_v, xbuf0, xbuf1,
          obuf0, obuf1, isem, osem):
        xbufs = [xbuf0, xbuf1]
        obufs = [obuf0, obuf1]
        wid = lax.axis_index("s") * NC + lax.axis_index("c")
        pltpu.sync_copy(ri_hbm, idx_v)
        pltpu.sync_copy(vm_hbm, vm_v)
        base = wid * ROWS_PER_W

        def in_copy(ci, b):
            row0 = base + ci * R
            return pltpu.make_async_copy(
                x_hbm.at[pl.ds(row0, R), :], xbufs[b], isem.at[b]
            )

        def out_copy(ci, b):
            row0 = base + ci * R
            return pltpu.make_async_copy(
                obufs[b], out_hbm.at[pl.ds(row0, R), :], osem.at[b]
            )

        for b in range(NBUF):
            in_copy(b, b).start()

        row_vecs = [jnp.full((L,), r, jnp.int32) for r in range(R)]

        def step_body(ci2, carry):
            for b in range(NBUF):
                ci = ci2 * NBUF + b
                in_copy(ci, b).wait()
                @pl.when(ci >= NBUF)
                def _():
                    out_copy(ci, b).wait()

                @plsc.parallel_loop(0, G, unroll=4)
                def _(g):
                    goff = g * L
                    idx16 = idx_v[pl.ds(goff, L)]
                    m16 = vm_v[pl.ds(goff, L)]
                    for r in range(R):
                        vals = plsc.load_gather(xbufs[b], [row_vecs[r], idx16])
                        obufs[b][r, pl.ds(goff, L)] = vals * m16
                out_copy(ci, b).start()

                @pl.when(ci + NBUF < CHUNKS)
                def _():
                    in_copy(ci + NBUF, b).start()

            return carry

        lax.fori_loop(0, CHUNKS // NBUF, step_body, 0, unroll=1)

        for b in range(NBUF):
            out_copy(CHUNKS - NBUF + b, b).wait()

    return k(x, routing_indices, valid_mask)
